# baseline (speedup 1.0000x reference)
import functools

import jax
import jax.numpy as jnp
from jax.experimental import pallas as pl
from jax.experimental.pallas import tpu as pltpu


def _se_kernel(x_ref, wd_ref, bd_ref, wu_ref, bu_ref, o_ref, *, inv_hw, nb):
    for b in range(nb):
        xs = x_ref[b]
        hw = xs.shape[1]
        ones = jnp.ones((hw, 8), jnp.float32)
        pooled = jax.lax.dot_general(
            xs, ones, (((1,), (0,)), ((), ())),
            preferred_element_type=jnp.float32) * inv_hw
        h = jnp.dot(wd_ref[...], pooled,
                    preferred_element_type=jnp.float32) + bd_ref[...]
        h = jnp.maximum(h, 0.0)
        z = jnp.dot(wu_ref[...], h,
                    preferred_element_type=jnp.float32) + bu_ref[...]
        s = jax.nn.sigmoid(z[:, 0:1])
        o_ref[b] = xs * s


@jax.jit
def _se_forward(x_nchw, down_w, down_b, up_w, up_b):
    n, c, h, w = x_nchw.shape
    internal = down_w.shape[0]
    hw = h * w
    x = x_nchw.reshape(n, c, hw)

    wd = down_w.astype(jnp.float32)
    bd = down_b.astype(jnp.float32).reshape(internal, 1)
    wu = up_w.astype(jnp.float32)
    bu = up_b.astype(jnp.float32).reshape(c, 1)

    nb = 1
    y = pl.pallas_call(
        functools.partial(_se_kernel, inv_hw=1.0 / hw, nb=nb),
        out_shape=jax.ShapeDtypeStruct((n, c, hw), x.dtype),
        grid=(n // nb,),
        in_specs=[
            pl.BlockSpec((nb, c, hw), lambda b: (b, 0, 0)),
            pl.BlockSpec((internal, c), lambda b: (0, 0)),
            pl.BlockSpec((internal, 1), lambda b: (0, 0)),
            pl.BlockSpec((c, internal), lambda b: (0, 0)),
            pl.BlockSpec((c, 1), lambda b: (0, 0)),
        ],
        out_specs=pl.BlockSpec((nb, c, hw), lambda b: (b, 0, 0)),
        compiler_params=pltpu.CompilerParams(
            dimension_semantics=("parallel",),
            vmem_limit_bytes=48 << 20),
    )(x, wd, bd, wu, bu)
    return y.reshape(n, c, h, w)


def kernel(x_nchw, down_w, down_b, up_w, up_b):
    return _se_forward(x_nchw, down_w, down_b, up_w, up_b)


# speedup vs baseline: 1.2482x; 1.2482x over previous
import functools

import jax
import jax.numpy as jnp
from jax.experimental import pallas as pl
from jax.experimental.pallas import tpu as pltpu


def _se_kernel(x_ref, wd_ref, bd_ref, wu_ref, bu_ref, o_ref, *, inv_hw, nb):
    for b in range(nb):
        xs = x_ref[b]
        pooled = jnp.sum(xs, axis=-1, keepdims=True) * inv_hw
        pc = jnp.broadcast_to(pooled, (pooled.shape[0], 8))
        h = jnp.dot(wd_ref[...], pc,
                    preferred_element_type=jnp.float32) + bd_ref[...]
        h = jnp.maximum(h, 0.0)
        z = jnp.dot(wu_ref[...], h,
                    preferred_element_type=jnp.float32) + bu_ref[...]
        s = jax.nn.sigmoid(z[:, 0:1])
        o_ref[b] = xs * s


@jax.jit
def _se_forward(x_nchw, down_w, down_b, up_w, up_b):
    n, c, h, w = x_nchw.shape
    internal = down_w.shape[0]
    hw = h * w
    x = x_nchw.reshape(n, c, hw)

    wd = down_w.astype(jnp.float32)
    bd = down_b.astype(jnp.float32).reshape(internal, 1)
    wu = up_w.astype(jnp.float32)
    bu = up_b.astype(jnp.float32).reshape(c, 1)

    nb = 8
    y = pl.pallas_call(
        functools.partial(_se_kernel, inv_hw=1.0 / hw, nb=nb),
        out_shape=jax.ShapeDtypeStruct((n, c, hw), x.dtype),
        grid=(n // nb,),
        in_specs=[
            pl.BlockSpec((nb, c, hw), lambda b: (b, 0, 0)),
            pl.BlockSpec((internal, c), lambda b: (0, 0)),
            pl.BlockSpec((internal, 1), lambda b: (0, 0)),
            pl.BlockSpec((c, internal), lambda b: (0, 0)),
            pl.BlockSpec((c, 1), lambda b: (0, 0)),
        ],
        out_specs=pl.BlockSpec((nb, c, hw), lambda b: (b, 0, 0)),
        compiler_params=pltpu.CompilerParams(
            dimension_semantics=("parallel",),
            vmem_limit_bytes=60 << 20),
    )(x, wd, bd, wu, bu)
    return y.reshape(n, c, h, w)


def kernel(x_nchw, down_w, down_b, up_w, up_b):
    return _se_forward(x_nchw, down_w, down_b, up_w, up_b)


# speedup vs baseline: 5.1855x; 4.1544x over previous
import functools

import jax
import jax.numpy as jnp
from jax.experimental import pallas as pl
from jax.experimental.pallas import tpu as pltpu


def _se_kernel(x_ref, wdt_ref, bd_ref, wut_ref, bu_ref, o_ref, *, inv_hw):
    xs = x_ref[...]
    pooled = jnp.sum(xs, axis=0) * inv_hw
    h = jnp.dot(pooled, wdt_ref[...],
                preferred_element_type=jnp.float32) + bd_ref[...]
    h = jnp.maximum(h, 0.0)
    z = jnp.dot(h, wut_ref[...],
                preferred_element_type=jnp.float32) + bu_ref[...]
    s = jax.nn.sigmoid(z)
    o_ref[...] = xs * s[None, :, :]


@jax.jit
def _se_forward(x_nchw, down_w, down_b, up_w, up_b):
    n, c, h, w = x_nchw.shape
    internal = down_w.shape[0]
    hw = h * w
    xt = x_nchw.reshape(n, c, hw).transpose(2, 0, 1)

    wdt = down_w.astype(jnp.float32).T
    bd = down_b.astype(jnp.float32).reshape(1, internal)
    wut = up_w.astype(jnp.float32).T
    bu = up_b.astype(jnp.float32).reshape(1, c)

    nb = 8 if n % 8 == 0 else 1
    y = pl.pallas_call(
        functools.partial(_se_kernel, inv_hw=1.0 / hw),
        out_shape=jax.ShapeDtypeStruct((hw, n, c), x_nchw.dtype),
        grid=(n // nb,),
        in_specs=[
            pl.BlockSpec((hw, nb, c), lambda b: (0, b, 0)),
            pl.BlockSpec((c, internal), lambda b: (0, 0)),
            pl.BlockSpec((1, internal), lambda b: (0, 0)),
            pl.BlockSpec((internal, c), lambda b: (0, 0)),
            pl.BlockSpec((1, c), lambda b: (0, 0)),
        ],
        out_specs=pl.BlockSpec((hw, nb, c), lambda b: (0, b, 0)),
        compiler_params=pltpu.CompilerParams(
            dimension_semantics=("arbitrary",),
            vmem_limit_bytes=60 << 20),
    )(xt, wdt, bd, wut, bu)
    return y.transpose(1, 2, 0).reshape(n, c, h, w)


def kernel(x_nchw, down_w, down_b, up_w, up_b):
    return _se_forward(x_nchw, down_w, down_b, up_w, up_b)


# speedup vs baseline: 5.7260x; 1.1042x over previous
import functools

import jax
import jax.numpy as jnp
from jax.experimental import pallas as pl
from jax.experimental.pallas import tpu as pltpu


def _se_kernel(x_ref, wdt_ref, bd_ref, wut_ref, bu_ref, o_ref, *, inv_hw):
    xs = x_ref[...]
    pooled = jnp.sum(xs, axis=0) * inv_hw
    h = jnp.dot(pooled, wdt_ref[...],
                preferred_element_type=jnp.float32) + bd_ref[...]
    h = jnp.maximum(h, 0.0)
    z = jnp.dot(h, wut_ref[...],
                preferred_element_type=jnp.float32) + bu_ref[...]
    s = jax.nn.sigmoid(z)
    o_ref[...] = xs * s[None, :, :]


@jax.jit
def _se_forward(x_nchw, down_w, down_b, up_w, up_b):
    n, c, h, w = x_nchw.shape
    internal = down_w.shape[0]
    hw = h * w
    xt = x_nchw.reshape(n, c, hw).transpose(2, 0, 1)

    wdt = down_w.astype(jnp.float32).T
    bd = down_b.astype(jnp.float32).reshape(1, internal)
    wut = up_w.astype(jnp.float32).T
    bu = up_b.astype(jnp.float32).reshape(1, c)

    nb = 16 if n % 16 == 0 else 1
    y = pl.pallas_call(
        functools.partial(_se_kernel, inv_hw=1.0 / hw),
        out_shape=jax.ShapeDtypeStruct((hw, n, c), x_nchw.dtype),
        grid=(n // nb,),
        in_specs=[
            pl.BlockSpec((hw, nb, c), lambda b: (0, b, 0)),
            pl.BlockSpec((c, internal), lambda b: (0, 0)),
            pl.BlockSpec((1, internal), lambda b: (0, 0)),
            pl.BlockSpec((internal, c), lambda b: (0, 0)),
            pl.BlockSpec((1, c), lambda b: (0, 0)),
        ],
        out_specs=pl.BlockSpec((hw, nb, c), lambda b: (0, b, 0)),
        compiler_params=pltpu.CompilerParams(
            dimension_semantics=("arbitrary",),
            vmem_limit_bytes=60 << 20),
    )(xt, wdt, bd, wut, bu)
    return y.transpose(1, 2, 0).reshape(n, c, h, w)


def kernel(x_nchw, down_w, down_b, up_w, up_b):
    return _se_forward(x_nchw, down_w, down_b, up_w, up_b)
